# baseline (speedup 1.0000x reference)
import functools

import jax
import jax.numpy as jnp
from jax import lax
from jax.experimental import pallas as pl
from jax.experimental.pallas import tpu as pltpu
from jax.experimental.pallas import tpu_sc as plsc

_N = 2_000_000
_H, _W = 720, 1280
_HW = _H * _W
_TOTAL = 2 * _HW
_NC = 2
_NS = 16
_NW = _NC * _NS
_SLICE = _TOTAL // _NW
_RC = 4000
_NCH = _N // _RC
_TPT = -(-_NCH // _NW)
_AC = 2048
_AT = 32
_BLK = 256
_RING = 2 * _BLK
_CAP = 64_000
_RPAD = _NW * _NW * _CAP
_DCHUNK = 7200
_TAU0, _TAU1 = 10000.0, 100000.0
_CELLS = _NW * _NS
_MAGIC = 9321


def _dyn_gather(x, i):
    dnums = lax.GatherDimensionNumbers(
        offset_dims=(), collapsed_slice_dims=(0,), start_index_map=(0,))
    return lax.gather(x, i[:, None], dnums, (1,),
                      mode=lax.GatherScatterMode.PROMISE_IN_BOUNDS)


def _extract(v, j):
    lane = lax.broadcasted_iota(jnp.int32, (16,), 0)
    return jnp.max(jnp.where(lane == j, v, 0))


def _wid():
    return lax.axis_index("s") * _NC + lax.axis_index("c")


def _flat_idx(xv, yv, pv):
    return (jnp.where(pv > 0, jnp.int32(_HW), jnp.int32(0))
            + yv * jnp.int32(_W) + xv)


def _owner(idx):
    return lax.shift_right_logical(
        lax.shift_right_logical(idx, 8) * jnp.int32(_MAGIC), 21)


def _route_body(x_hbm, y_hbm, p_hbm, t_hbm, ridx_hbm, rt_hbm, cnt_hbm,
                fill, sidx, st, evx, evy, evp, evt, fl_s, pend_s, sem,
                insem):
    w = _wid()
    lane = lax.broadcasted_iota(jnp.int32, (16,), 0)

    fill[pl.ds(0, 16)] = jnp.zeros((16,), jnp.int32)
    fill[pl.ds(16, 16)] = jnp.zeros((16,), jnp.int32)

    def zs(i, _):
        fl_s[i] = 0
        return 0
    lax.fori_loop(0, _NW, zs, 0, unroll=False)
    pend_s[0] = 0

    def flush_block(o, final):
        fo = _extract(fill[pl.ds((o // 16) * 16, 16)], o % 16)
        blk = fl_s[o]
        ready = (fo >> 8) > blk
        if final:
            ready = fo > blk * _BLK

        @pl.when(ready)
        def _():
            @pl.when(pend_s[0] > 0)
            def _():
                pltpu.make_async_copy(
                    ridx_hbm.at[pl.ds(0, _BLK)], sidx.at[pl.ds(0, _BLK)],
                    sem).wait()
                pltpu.make_async_copy(
                    ridx_hbm.at[pl.ds(0, _BLK)], sidx.at[pl.ds(0, _BLK)],
                    sem).wait()
            src = pl.multiple_of(o * _RING + (blk & 1) * _BLK, _BLK)
            dst = pl.multiple_of((o * _NW + w) * _CAP + blk * _BLK, _BLK)
            pltpu.async_copy(sidx.at[pl.ds(src, _BLK)],
                             ridx_hbm.at[pl.ds(dst, _BLK)], sem)
            pltpu.async_copy(st.at[pl.ds(src, _BLK)],
                             rt_hbm.at[pl.ds(dst, _BLK)], sem)
            fl_s[o] = blk + 1
            pend_s[0] = 1

    def turn(it, _):
        ci = it * _NW + w

        @pl.when(ci < _NCH)
        def _():
            b = ci * _RC
            ca = pltpu.async_copy(x_hbm.at[pl.ds(b, _RC)], evx, insem)
            cb = pltpu.async_copy(y_hbm.at[pl.ds(b, _RC)], evy, insem)
            cc = pltpu.async_copy(p_hbm.at[pl.ds(b, _RC)], evp, insem)
            cd = pltpu.async_copy(t_hbm.at[pl.ds(b, _RC)], evt, insem)
            ca.wait()
            cb.wait()
            cc.wait()
            cd.wait()

            def vec(vi, _):
                o = vi * 16
                idxv = _flat_idx(evx[pl.ds(o, 16)], evy[pl.ds(o, 16)],
                                 evp[pl.ds(o, 16)])
                tv = evt[pl.ds(o, 16)]
                sk, sv = plsc.sort_key_val(_owner(idxv), lane)
                idx_s = _dyn_gather(idxv, sv)
                t_s = _dyn_gather(tv, sv)
                pk = _dyn_gather(sk, jnp.maximum(lane - 1, 0))
                start = (lane == 0) | (sk != pk)
                startidx = plsc.cummax(jnp.where(start, lane, 0))
                rank = lane - startidx
                base = plsc.load_gather(fill, [sk])
                newf = base + rank + 1
                nk = _dyn_gather(sk, jnp.minimum(lane + 1, 15))
                runlast = (lane == 15) | (sk != nk)
                plsc.store_scatter(fill, [sk], newf, mask=runlast)
                slot = sk * jnp.int32(_RING) + \
                    lax.bitwise_and(base + rank, jnp.int32(_RING - 1))
                plsc.store_scatter(sidx, [slot], idx_s)
                plsc.store_scatter(st, [slot], t_s)
                crossed = runlast & ((newf >> 8) > (base >> 8))

                @pl.when(jnp.any(crossed))
                def _():
                    for o32 in range(_NW):
                        flush_block(o32, final=False)
                return 0
            lax.fori_loop(0, _RC // 16, vec, 0, unroll=2)
        return 0
    lax.fori_loop(0, _TPT, turn, 0, unroll=False)

    for o32 in range(_NW):
        flush_block(o32, final=True)

    @pl.when(pend_s[0] > 0)
    def _():
        pltpu.make_async_copy(
            ridx_hbm.at[pl.ds(0, _BLK)], sidx.at[pl.ds(0, _BLK)], sem).wait()
        pltpu.make_async_copy(
            ridx_hbm.at[pl.ds(0, _BLK)], sidx.at[pl.ds(0, _BLK)], sem).wait()

    pltpu.sync_copy(fill, cnt_hbm.at[pl.ds(w * _NW, _NW)])


_route = functools.partial(
    pl.kernel,
    mesh=plsc.VectorSubcoreMesh(core_axis_name="c", subcore_axis_name="s"),
    compiler_params=pltpu.CompilerParams(needs_layout_passes=False),
    out_type=[
        jax.ShapeDtypeStruct((_RPAD,), jnp.int32),
        jax.ShapeDtypeStruct((_RPAD,), jnp.float32),
        jax.ShapeDtypeStruct((_NW * _NW,), jnp.int32),
    ],
    scratch_types=[
        pltpu.VMEM((_NW,), jnp.int32),
        pltpu.VMEM((_NW * _RING,), jnp.int32),
        pltpu.VMEM((_NW * _RING,), jnp.float32),
        pltpu.VMEM((_RC,), jnp.int32),
        pltpu.VMEM((_RC,), jnp.int32),
        pltpu.VMEM((_RC,), jnp.int32),
        pltpu.VMEM((_RC,), jnp.float32),
        pltpu.SMEM((_NW,), jnp.int32),
        pltpu.SMEM((2,), jnp.int32),
        pltpu.SemaphoreType.DMA,
        pltpu.SemaphoreType.DMA,
    ],
)(_route_body)


def _apply_body(ridx_hbm, rt_hbm, cnt_hbm, ts0_hbm, ct_hbm,
                surf_hbm, ts_hbm,
                cnts, slice_ref, evi, evt, d0, d1, d0b, d1b, ctv,
                insem, osem):
    w = _wid()
    lo = w * _SLICE
    pltpu.sync_copy(ts0_hbm.at[pl.ds(lo, _SLICE)], slice_ref)
    pltpu.sync_copy(ct_hbm, ctv)
    pltpu.sync_copy(cnt_hbm, cnts)

    lane = lax.broadcasted_iota(jnp.int32, (16,), 0)

    def seg(k, _):
        s0 = pl.multiple_of((w * _NW + k) * _CAP, _BLK)
        c = _extract(cnts[pl.ds(k * _NW + (w // 16) * 16, 16)], w & 15)

        def chunk(tk, _):
            off = tk * _AC

            @pl.when(off < c)
            def _():
                ca = pltpu.async_copy(ridx_hbm.at[pl.ds(s0 + off, _AC)],
                                      evi, insem)
                cb = pltpu.async_copy(rt_hbm.at[pl.ds(s0 + off, _AC)],
                                      evt, insem)
                ca.wait()
                cb.wait()

                def vec(vi, _):
                    o = vi * 16
                    act = (off + o + lane) < c
                    idxv = evi[pl.ds(o, 16)]
                    tv = evt[pl.ds(o, 16)]
                    loc = idxv - lo
                    locc = jnp.where(act, loc, 0)

                    key = jnp.where(act, locc, jnp.int32(_SLICE) + lane)
                    sk0, _sv0 = plsc.sort_key_val(key, lane)
                    pk0 = _dyn_gather(sk0, jnp.maximum(lane - 1, 0))
                    hasdup = jnp.any((lane > 0) & (sk0 == pk0))

                    cur = plsc.load_gather(slice_ref, [locc], mask=act)
                    need = act & (tv > cur)
                    plsc.store_scatter(slice_ref, [locc], tv, mask=need)

                    @pl.when(hasdup)
                    def _():
                        still = act
                        key = jnp.where(still, locc, jnp.int32(_SLICE))
                        sk, sv = plsc.sort_key_val(key, lane)
                        act_s = sk < _SLICE
                        ts_s = _dyn_gather(tv, sv)
                        pk = _dyn_gather(sk, jnp.maximum(lane - 1, 0))
                        sstart = (lane == 0) | (sk != pk)
                        sidx0 = plsc.cummax(jnp.where(sstart, lane, 0))
                        run = ts_s
                        for sh in (1, 2, 4, 8):
                            shf = _dyn_gather(run, jnp.maximum(lane - sh, 0))
                            ok = (lane - sh) >= sidx0
                            run = jnp.where(ok, jnp.maximum(run, shf), run)
                        nk = _dyn_gather(sk, jnp.minimum(lane + 1, 15))
                        rl = act_s & ((lane == 15) | (sk != nk))
                        locs = jnp.where(act_s, sk, 0)
                        cur3 = plsc.load_gather(slice_ref, [locs], mask=rl)
                        plsc.store_scatter(slice_ref, [locs],
                                           jnp.maximum(cur3, run), mask=rl)
                    return 0
                lax.fori_loop(0, _AC // 16, vec, 0, unroll=False)
            return 0
        lax.fori_loop(0, _AT, chunk, 0, unroll=False)
        return 0
    lax.fori_loop(0, _NW, seg, 0, unroll=False)

    pol = w // _NS
    s0_base = lo + pol * _HW
    tsc = pltpu.async_copy(slice_ref, ts_hbm.at[pl.ds(lo, _SLICE)], insem)

    bufs = ((d0, d1), (d0b, d1b))
    pending = [None, None]
    for di in range(_SLICE // _DCHUNK):
        off = di * _DCHUNK
        b0, b1 = bufs[di & 1]
        if pending[di & 1] is not None:
            for c in pending[di & 1]:
                c.wait()

        def dvec(vi, _, b0=b0, b1=b1, off=off):
            o = vi * 16
            ts = slice_ref[pl.ds(off + o, 16)]
            m = ts > 0.0
            z = ts - ctv[...]
            b0[pl.ds(o, 16)] = jnp.where(
                m, jnp.exp(z * jnp.float32(1.0 / _TAU0)), 0.0)
            b1[pl.ds(o, 16)] = jnp.where(
                m, jnp.exp(z * jnp.float32(1.0 / _TAU1)), 0.0)
            return 0
        lax.fori_loop(0, _DCHUNK // 16, dvec, 0, unroll=False)
        pending[di & 1] = (
            pltpu.async_copy(
                b0, surf_hbm.at[pl.ds(s0_base + off, _DCHUNK)], osem),
            pltpu.async_copy(
                b1, surf_hbm.at[pl.ds(s0_base + _HW + off, _DCHUNK)], osem),
        )
    for pend in pending:
        if pend is not None:
            for c in pend:
                c.wait()
    tsc.wait()


_apply = functools.partial(
    pl.kernel,
    mesh=plsc.VectorSubcoreMesh(core_axis_name="c", subcore_axis_name="s"),
    compiler_params=pltpu.CompilerParams(needs_layout_passes=False),
    out_type=[
        jax.ShapeDtypeStruct((4 * _HW,), jnp.float32),
        jax.ShapeDtypeStruct((2 * _HW,), jnp.float32),
    ],
    scratch_types=[
        pltpu.VMEM((_NW * _NW,), jnp.int32),
        pltpu.VMEM((_SLICE,), jnp.float32),
        pltpu.VMEM((_AC,), jnp.int32),
        pltpu.VMEM((_AC,), jnp.float32),
        pltpu.VMEM((_DCHUNK,), jnp.float32),
        pltpu.VMEM((_DCHUNK,), jnp.float32),
        pltpu.VMEM((_DCHUNK,), jnp.float32),
        pltpu.VMEM((_DCHUNK,), jnp.float32),
        pltpu.VMEM((16,), jnp.float32),
        pltpu.SemaphoreType.DMA,
        pltpu.SemaphoreType.DMA,
    ],
)(_apply_body)


def kernel(t, x, y, p, time_stamp, curr_time):
    x32 = x.astype(jnp.int32)
    y32 = y.astype(jnp.int32)
    p32 = p.astype(jnp.int32)
    ts0 = time_stamp.reshape(-1)
    ctv = jnp.full((16,), curr_time, dtype=jnp.float32)
    ridx, rt, cnt_all = _route(x32, y32, p32, t)
    surf_flat, ts_flat = _apply(ridx, rt, cnt_all, ts0, ctv)
    return surf_flat.reshape(4, _H, _W), ts_flat.reshape(2, _H, _W)


# speedup vs baseline: 1.0527x; 1.0527x over previous
import functools

import jax
import jax.numpy as jnp
from jax import lax
from jax.experimental import pallas as pl
from jax.experimental.pallas import tpu as pltpu
from jax.experimental.pallas import tpu_sc as plsc

_N = 2_000_000
_H, _W = 720, 1280
_HW = _H * _W
_TOTAL = 2 * _HW
_NC = 2
_NS = 16
_NW = _NC * _NS
_SLICE = _TOTAL // _NW
_RC = 8000
_NCH = _N // _RC
_TPT = -(-_NCH // _NW)
_AC = 2048
_AT = 32
_BLK = 256
_RING = 2 * _BLK
_CAP = 64_000
_RPAD = _NW * _NW * _CAP
_DCHUNK = 7200
_TAU0, _TAU1 = 10000.0, 100000.0
_CELLS = _NW * _NS
_MAGIC = 9321


def _dyn_gather(x, i):
    dnums = lax.GatherDimensionNumbers(
        offset_dims=(), collapsed_slice_dims=(0,), start_index_map=(0,))
    return lax.gather(x, i[:, None], dnums, (1,),
                      mode=lax.GatherScatterMode.PROMISE_IN_BOUNDS)


def _extract(v, j):
    lane = lax.broadcasted_iota(jnp.int32, (16,), 0)
    return jnp.max(jnp.where(lane == j, v, 0))


def _wid():
    return lax.axis_index("s") * _NC + lax.axis_index("c")


def _flat_idx(xv, yv, pv):
    return (jnp.where(pv > 0, jnp.int32(_HW), jnp.int32(0))
            + yv * jnp.int32(_W) + xv)


def _owner(idx):
    return lax.shift_right_logical(
        lax.shift_right_logical(idx, 8) * jnp.int32(_MAGIC), 21)


def _route_body(x_hbm, y_hbm, p_hbm, t_hbm, ridx_hbm, rt_hbm, cnt_hbm,
                fill, sidx, st, evx, evy, evp, evt, fl_s, pend_s, sem,
                insem):
    w = _wid()
    lane = lax.broadcasted_iota(jnp.int32, (16,), 0)

    fill[pl.ds(0, 16)] = jnp.zeros((16,), jnp.int32)
    fill[pl.ds(16, 16)] = jnp.zeros((16,), jnp.int32)

    def zs(i, _):
        fl_s[i] = 0
        return 0
    lax.fori_loop(0, _NW, zs, 0, unroll=False)
    pend_s[0] = 0

    def flush_block(o, final):
        fo = _extract(fill[pl.ds((o // 16) * 16, 16)], o % 16)
        blk = fl_s[o]
        ready = (fo >> 8) > blk
        if final:
            ready = fo > blk * _BLK

        @pl.when(ready)
        def _():
            @pl.when(pend_s[0] > 0)
            def _():
                pltpu.make_async_copy(
                    ridx_hbm.at[pl.ds(0, _BLK)], sidx.at[pl.ds(0, _BLK)],
                    sem).wait()
                pltpu.make_async_copy(
                    ridx_hbm.at[pl.ds(0, _BLK)], sidx.at[pl.ds(0, _BLK)],
                    sem).wait()
            src = pl.multiple_of(o * _RING + (blk & 1) * _BLK, _BLK)
            dst = pl.multiple_of((o * _NW + w) * _CAP + blk * _BLK, _BLK)
            pltpu.async_copy(sidx.at[pl.ds(src, _BLK)],
                             ridx_hbm.at[pl.ds(dst, _BLK)], sem)
            pltpu.async_copy(st.at[pl.ds(src, _BLK)],
                             rt_hbm.at[pl.ds(dst, _BLK)], sem)
            fl_s[o] = blk + 1
            pend_s[0] = 1

    def turn(it, _):
        ci = it * _NW + w

        @pl.when(ci < _NCH)
        def _():
            b = ci * _RC
            ca = pltpu.async_copy(x_hbm.at[pl.ds(b, _RC)], evx, insem)
            cb = pltpu.async_copy(y_hbm.at[pl.ds(b, _RC)], evy, insem)
            cc = pltpu.async_copy(p_hbm.at[pl.ds(b, _RC)], evp, insem)
            cd = pltpu.async_copy(t_hbm.at[pl.ds(b, _RC)], evt, insem)
            ca.wait()
            cb.wait()
            cc.wait()
            cd.wait()

            def vec(vi, _):
                o = vi * 16
                idxv = _flat_idx(evx[pl.ds(o, 16)], evy[pl.ds(o, 16)],
                                 evp[pl.ds(o, 16)])
                tv = evt[pl.ds(o, 16)]
                sk, sv = plsc.sort_key_val(_owner(idxv), lane)
                idx_s = _dyn_gather(idxv, sv)
                t_s = _dyn_gather(tv, sv)
                pk = _dyn_gather(sk, jnp.maximum(lane - 1, 0))
                start = (lane == 0) | (sk != pk)
                startidx = plsc.cummax(jnp.where(start, lane, 0))
                rank = lane - startidx
                base = plsc.load_gather(fill, [sk])
                newf = base + rank + 1
                nk = _dyn_gather(sk, jnp.minimum(lane + 1, 15))
                runlast = (lane == 15) | (sk != nk)
                plsc.store_scatter(fill, [sk], newf, mask=runlast)
                slot = sk * jnp.int32(_RING) + \
                    lax.bitwise_and(base + rank, jnp.int32(_RING - 1))
                plsc.store_scatter(sidx, [slot], idx_s)
                plsc.store_scatter(st, [slot], t_s)
                crossed = runlast & ((newf >> 8) > (base >> 8))

                @pl.when(jnp.any(crossed))
                def _():
                    for o32 in range(_NW):
                        flush_block(o32, final=False)
                return 0
            lax.fori_loop(0, _RC // 16, vec, 0, unroll=False)
        return 0
    lax.fori_loop(0, _TPT, turn, 0, unroll=False)

    for o32 in range(_NW):
        flush_block(o32, final=True)

    @pl.when(pend_s[0] > 0)
    def _():
        pltpu.make_async_copy(
            ridx_hbm.at[pl.ds(0, _BLK)], sidx.at[pl.ds(0, _BLK)], sem).wait()
        pltpu.make_async_copy(
            ridx_hbm.at[pl.ds(0, _BLK)], sidx.at[pl.ds(0, _BLK)], sem).wait()

    pltpu.sync_copy(fill, cnt_hbm.at[pl.ds(w * _NW, _NW)])


_route = functools.partial(
    pl.kernel,
    mesh=plsc.VectorSubcoreMesh(core_axis_name="c", subcore_axis_name="s"),
    compiler_params=pltpu.CompilerParams(needs_layout_passes=False),
    out_type=[
        jax.ShapeDtypeStruct((_RPAD,), jnp.int32),
        jax.ShapeDtypeStruct((_RPAD,), jnp.float32),
        jax.ShapeDtypeStruct((_NW * _NW,), jnp.int32),
    ],
    scratch_types=[
        pltpu.VMEM((_NW,), jnp.int32),
        pltpu.VMEM((_NW * _RING,), jnp.int32),
        pltpu.VMEM((_NW * _RING,), jnp.float32),
        pltpu.VMEM((_RC,), jnp.int32),
        pltpu.VMEM((_RC,), jnp.int32),
        pltpu.VMEM((_RC,), jnp.int32),
        pltpu.VMEM((_RC,), jnp.float32),
        pltpu.SMEM((_NW,), jnp.int32),
        pltpu.SMEM((2,), jnp.int32),
        pltpu.SemaphoreType.DMA,
        pltpu.SemaphoreType.DMA,
    ],
)(_route_body)


def _apply_body(ridx_hbm, rt_hbm, cnt_hbm, ts0_hbm, ct_hbm,
                surf_hbm, ts_hbm,
                cnts, slice_ref, evi, evt, d0, d1, d0b, d1b, ctv,
                insem, osem):
    w = _wid()
    lo = w * _SLICE
    pltpu.sync_copy(ts0_hbm.at[pl.ds(lo, _SLICE)], slice_ref)
    pltpu.sync_copy(ct_hbm, ctv)
    pltpu.sync_copy(cnt_hbm, cnts)

    lane = lax.broadcasted_iota(jnp.int32, (16,), 0)

    def seg(k, _):
        s0 = pl.multiple_of((w * _NW + k) * _CAP, _BLK)
        c = _extract(cnts[pl.ds(k * _NW + (w // 16) * 16, 16)], w & 15)

        def chunk(tk, _):
            off = tk * _AC

            @pl.when(off < c)
            def _():
                ca = pltpu.async_copy(ridx_hbm.at[pl.ds(s0 + off, _AC)],
                                      evi, insem)
                cb = pltpu.async_copy(rt_hbm.at[pl.ds(s0 + off, _AC)],
                                      evt, insem)
                ca.wait()
                cb.wait()

                def vec(vi, _):
                    o = vi * 16
                    act = (off + o + lane) < c
                    idxv = evi[pl.ds(o, 16)]
                    tv = evt[pl.ds(o, 16)]
                    loc = idxv - lo
                    locc = jnp.where(act, loc, 0)

                    key = jnp.where(act, locc, jnp.int32(_SLICE) + lane)
                    sk0, _sv0 = plsc.sort_key_val(key, lane)
                    pk0 = _dyn_gather(sk0, jnp.maximum(lane - 1, 0))
                    hasdup = jnp.any((lane > 0) & (sk0 == pk0))

                    cur = plsc.load_gather(slice_ref, [locc], mask=act)
                    need = act & (tv > cur)
                    plsc.store_scatter(slice_ref, [locc], tv, mask=need)

                    @pl.when(hasdup)
                    def _():
                        still = act
                        key = jnp.where(still, locc, jnp.int32(_SLICE))
                        sk, sv = plsc.sort_key_val(key, lane)
                        act_s = sk < _SLICE
                        ts_s = _dyn_gather(tv, sv)
                        pk = _dyn_gather(sk, jnp.maximum(lane - 1, 0))
                        sstart = (lane == 0) | (sk != pk)
                        sidx0 = plsc.cummax(jnp.where(sstart, lane, 0))
                        run = ts_s
                        for sh in (1, 2, 4, 8):
                            shf = _dyn_gather(run, jnp.maximum(lane - sh, 0))
                            ok = (lane - sh) >= sidx0
                            run = jnp.where(ok, jnp.maximum(run, shf), run)
                        nk = _dyn_gather(sk, jnp.minimum(lane + 1, 15))
                        rl = act_s & ((lane == 15) | (sk != nk))
                        locs = jnp.where(act_s, sk, 0)
                        cur3 = plsc.load_gather(slice_ref, [locs], mask=rl)
                        plsc.store_scatter(slice_ref, [locs],
                                           jnp.maximum(cur3, run), mask=rl)
                    return 0
                lax.fori_loop(0, _AC // 16, vec, 0, unroll=False)
            return 0
        lax.fori_loop(0, _AT, chunk, 0, unroll=False)
        return 0
    lax.fori_loop(0, _NW, seg, 0, unroll=False)

    pol = w // _NS
    s0_base = lo + pol * _HW
    tsc = pltpu.async_copy(slice_ref, ts_hbm.at[pl.ds(lo, _SLICE)], insem)

    bufs = ((d0, d1), (d0b, d1b))
    pending = [None, None]
    for di in range(_SLICE // _DCHUNK):
        off = di * _DCHUNK
        b0, b1 = bufs[di & 1]
        if pending[di & 1] is not None:
            for c in pending[di & 1]:
                c.wait()

        def dvec(vi, _, b0=b0, b1=b1, off=off):
            o = vi * 16
            ts = slice_ref[pl.ds(off + o, 16)]
            m = ts > 0.0
            z = ts - ctv[...]
            b0[pl.ds(o, 16)] = jnp.where(
                m, jnp.exp(z * jnp.float32(1.0 / _TAU0)), 0.0)
            b1[pl.ds(o, 16)] = jnp.where(
                m, jnp.exp(z * jnp.float32(1.0 / _TAU1)), 0.0)
            return 0
        lax.fori_loop(0, _DCHUNK // 16, dvec, 0, unroll=False)
        pending[di & 1] = (
            pltpu.async_copy(
                b0, surf_hbm.at[pl.ds(s0_base + off, _DCHUNK)], osem),
            pltpu.async_copy(
                b1, surf_hbm.at[pl.ds(s0_base + _HW + off, _DCHUNK)], osem),
        )
    for pend in pending:
        if pend is not None:
            for c in pend:
                c.wait()
    tsc.wait()


_apply = functools.partial(
    pl.kernel,
    mesh=plsc.VectorSubcoreMesh(core_axis_name="c", subcore_axis_name="s"),
    compiler_params=pltpu.CompilerParams(needs_layout_passes=False),
    out_type=[
        jax.ShapeDtypeStruct((4 * _HW,), jnp.float32),
        jax.ShapeDtypeStruct((2 * _HW,), jnp.float32),
    ],
    scratch_types=[
        pltpu.VMEM((_NW * _NW,), jnp.int32),
        pltpu.VMEM((_SLICE,), jnp.float32),
        pltpu.VMEM((_AC,), jnp.int32),
        pltpu.VMEM((_AC,), jnp.float32),
        pltpu.VMEM((_DCHUNK,), jnp.float32),
        pltpu.VMEM((_DCHUNK,), jnp.float32),
        pltpu.VMEM((_DCHUNK,), jnp.float32),
        pltpu.VMEM((_DCHUNK,), jnp.float32),
        pltpu.VMEM((16,), jnp.float32),
        pltpu.SemaphoreType.DMA,
        pltpu.SemaphoreType.DMA,
    ],
)(_apply_body)


def kernel(t, x, y, p, time_stamp, curr_time):
    x32 = x.astype(jnp.int32)
    y32 = y.astype(jnp.int32)
    p32 = p.astype(jnp.int32)
    ts0 = time_stamp.reshape(-1)
    ctv = jnp.full((16,), curr_time, dtype=jnp.float32)
    ridx, rt, cnt_all = _route(x32, y32, p32, t)
    surf_flat, ts_flat = _apply(ridx, rt, cnt_all, ts0, ctv)
    return surf_flat.reshape(4, _H, _W), ts_flat.reshape(2, _H, _W)
